# baseline (speedup 1.0000x reference)
import functools

import jax
import jax.numpy as jnp
from jax import lax
from jax.experimental import pallas as pl
from jax.experimental.pallas import tpu as pltpu
from jax.experimental.pallas import tpu_sc as plsc

_VOCAB = 1000
_BATCH = 4096
_HIST = 20
_D = _VOCAB
_B = _BATCH * _HIST

_NC = 2
_NS = 16
_NW = _NC * _NS
_BPW = _B // _NW
_C = 32
_NCHUNK = _BPW // _C

_mesh = plsc.VectorSubcoreMesh(
    core_axis_name="c", subcore_axis_name="s", num_cores=_NC, num_subcores=_NS
)


@functools.partial(
    pl.kernel,
    out_type=jax.ShapeDtypeStruct((_B, _D), jnp.float32),
    mesh=_mesh,
    scratch_types=[
        pltpu.VMEM((_BPW,), jnp.int32),
        pltpu.VMEM((_C, _D), jnp.float32),
        pltpu.VMEM((_C, _D), jnp.float32),
        pltpu.VMEM((_C, _D), jnp.float32),
        pltpu.VMEM((_C, _D), jnp.float32),
        pltpu.SemaphoreType.DMA,
        pltpu.SemaphoreType.DMA,
        pltpu.SemaphoreType.DMA,
        pltpu.SemaphoreType.DMA,
        pltpu.SemaphoreType.DMA,
        pltpu.SemaphoreType.DMA,
        pltpu.SemaphoreType.DMA,
        pltpu.SemaphoreType.DMA,
    ],
    compiler_params=pltpu.CompilerParams(use_tc_tiling_on_sc=False),
)
def _embed_lookup(idx_hbm, table_hbm, out_hbm, idx_v, b0, b1, b2, b3,
                  g0, g1, g2, g3, w0, w1, w2, w3):
    bufs = [b0, b1, b2, b3]
    gsems = [g0, g1, g2, g3]
    wsems = [w0, w1, w2, w3]
    wid = lax.axis_index("s") * _NC + lax.axis_index("c")
    base = wid * _BPW
    pltpu.sync_copy(idx_hbm.at[pl.ds(base, _BPW)], idx_v)

    def gather_start(chunk, j):
        idx_slice = idx_v.at[pl.ds(chunk * _C, _C)]
        pltpu.async_copy(table_hbm.at[idx_slice], bufs[j], gsems[j])

    def gather_wait(chunk, j):
        idx_slice = idx_v.at[pl.ds(chunk * _C, _C)]
        pltpu.make_async_copy(table_hbm.at[idx_slice], bufs[j], gsems[j]).wait()

    def write_start(chunk, j):
        pltpu.async_copy(bufs[j], out_hbm.at[pl.ds(base + chunk * _C, _C)], wsems[j])

    def write_wait(chunk, j):
        pltpu.make_async_copy(
            bufs[j], out_hbm.at[pl.ds(base + chunk * _C, _C)], wsems[j]
        ).wait()

    gather_start(0, 0)
    gather_start(1, 1)
    gather_start(2, 2)
    gather_wait(0, 0)
    write_start(0, 0)
    gather_start(3, 3)
    gather_wait(1, 1)
    write_start(1, 1)

    def body(t, carry):
        for k in range(4):
            c = 2 + 4 * t + k
            jn = k % 4
            write_wait(c - 2, jn)
            gather_start(c + 2, jn)
            j = (2 + k) % 4
            gather_wait(c, j)
            write_start(c, j)
        return carry

    lax.fori_loop(0, (_NCHUNK - 4) // 4, body, 0)

    for c in (_NCHUNK - 2, _NCHUNK - 1):
        write_wait(c - 2, (c + 2) % 4)
        gather_wait(c, c % 4)
        write_start(c, c % 4)
    write_wait(_NCHUNK - 2, (_NCHUNK - 2) % 4)
    write_wait(_NCHUNK - 1, (_NCHUNK - 1) % 4)


def kernel(x, token_embedding_weight):
    idx = x.reshape(-1).astype(jnp.int32)
    out = _embed_lookup(idx, token_embedding_weight)
    return out.reshape(_BATCH, _HIST, _VOCAB)
